# Initial kernel scaffold; baseline (speedup 1.0000x reference)
import jax
import jax.numpy as jnp
from jax.experimental import pallas as pl


def kernel(atom_types, bond_types, atom_mask, atom_charges, atom_type_table, charge_table, bond_table, W1, b1, W2, b2):
    raise NotImplementedError("write your pallas kernel here")


# speedup vs baseline: 3.3881x; 3.3881x over previous
import jax
import jax.numpy as jnp
from jax import lax
from jax.experimental import pallas as pl

B, N = 256, 64
EMB = 64
D_INV = 256
ROWS = B * N
E_ROWS = B * N * N

INV_BLK = 2048
EDGE_BLK = 8192

A_VOCAB = 128
C_VOCAB = 8
B_VOCAB = 8


def _invs_body(at_ref, ac_ref, ta_ref, tc_ref, w1a_ref, w1c_ref, b1_ref,
               w2_ref, b2_ref, out_ref):
    at = at_ref[...]
    ac = ac_ref[...]
    ia = lax.broadcasted_iota(jnp.int32, (INV_BLK, A_VOCAB), 1)
    oh_a = (ia == at).astype(jnp.float32)
    emb_a = jnp.dot(oh_a, ta_ref[...], preferred_element_type=jnp.float32)
    ic = lax.broadcasted_iota(jnp.int32, (INV_BLK, C_VOCAB), 1)
    oh_c = (ic == ac).astype(jnp.float32)
    emb_c = jnp.dot(oh_c, tc_ref[...], preferred_element_type=jnp.float32)
    h = (jnp.dot(emb_a, w1a_ref[...], preferred_element_type=jnp.float32)
         + jnp.dot(emb_c, w1c_ref[...], preferred_element_type=jnp.float32)
         + b1_ref[...])
    h = h * jax.nn.sigmoid(h)
    out_ref[...] = (jnp.dot(h, w2_ref[...], preferred_element_type=jnp.float32)
                    + b2_ref[...])


def _edges_body(bt_ref, tbl_ref, out_ref):
    bt = bt_ref[...]
    io = lax.broadcasted_iota(jnp.int32, (EDGE_BLK, B_VOCAB), 1)
    oh = (io == bt).astype(jnp.float32)
    out_ref[...] = jnp.dot(oh, tbl_ref[...], preferred_element_type=jnp.float32)


def kernel(atom_types, bond_types, atom_mask, atom_charges, atom_type_table,
           charge_table, bond_table, W1, b1, W2, b2):
    del atom_mask

    at = atom_types.reshape(ROWS, 1)
    ac = atom_charges.reshape(ROWS, 1)
    ta = jnp.pad(atom_type_table, ((0, A_VOCAB - atom_type_table.shape[0]), (0, 0)))
    tc = jnp.pad(charge_table, ((0, C_VOCAB - charge_table.shape[0]), (0, 0)))
    w1a, w1c = W1[:EMB], W1[EMB:]
    b1r = b1.reshape(1, D_INV)
    b2r = b2.reshape(1, D_INV)

    full = lambda shape: pl.BlockSpec(shape, lambda i: (0,) * len(shape))
    invs = pl.pallas_call(
        _invs_body,
        grid=(ROWS // INV_BLK,),
        in_specs=[
            pl.BlockSpec((INV_BLK, 1), lambda i: (i, 0)),
            pl.BlockSpec((INV_BLK, 1), lambda i: (i, 0)),
            full((A_VOCAB, EMB)),
            full((C_VOCAB, EMB)),
            full((EMB, D_INV)),
            full((EMB, D_INV)),
            full((1, D_INV)),
            full((D_INV, D_INV)),
            full((1, D_INV)),
        ],
        out_specs=pl.BlockSpec((INV_BLK, D_INV), lambda i: (i, 0)),
        out_shape=jax.ShapeDtypeStruct((ROWS, D_INV), jnp.float32),
    )(at, ac, ta, tc, w1a, w1c, b1r, W2, b2r)

    bt = bond_types.reshape(E_ROWS, 1)
    edges = pl.pallas_call(
        _edges_body,
        grid=(E_ROWS // EDGE_BLK,),
        in_specs=[
            pl.BlockSpec((EDGE_BLK, 1), lambda i: (i, 0)),
            full((B_VOCAB, EMB)),
        ],
        out_specs=pl.BlockSpec((EDGE_BLK, EMB), lambda i: (i, 0)),
        out_shape=jax.ShapeDtypeStruct((E_ROWS, EMB), jnp.float32),
    )(bt, bond_table)

    return invs.reshape(B, N, D_INV), edges.reshape(B, N, N, EMB)


# speedup vs baseline: 9.0033x; 2.6573x over previous
import jax
import jax.numpy as jnp
from jax import lax
from jax.experimental import pallas as pl

B, N = 256, 64
EMB = 64
D_INV = 256
ROWS = B * N
E_ROWS = B * N * N

INV_BLK = 32
EDGE_BLK = 128

A_VOCAB = 128
C_VOCAB = 8
B_VOCAB = 8


def _onehot_t(idx_row, vocab):
    io = lax.broadcasted_iota(jnp.int32, (vocab, 64), 0)
    return (io == idx_row).astype(jnp.float32)


_TDOT = (((0,), (0,)), ((), ()))


def _invs_body(at_ref, ac_ref, ta_ref, tc_ref, w1a_ref, w1c_ref, b1_ref,
               w2_ref, b2_ref, out_ref):
    emb_a = []
    emb_c = []
    for i in range(INV_BLK):
        oh_a = _onehot_t(at_ref[i:i + 1, :], A_VOCAB)
        emb_a.append(lax.dot_general(oh_a, ta_ref[...], _TDOT,
                                     preferred_element_type=jnp.float32))
        oh_c = _onehot_t(ac_ref[i:i + 1, :], C_VOCAB)
        emb_c.append(lax.dot_general(oh_c, tc_ref[...], _TDOT,
                                     preferred_element_type=jnp.float32))
    ea = jnp.concatenate(emb_a, axis=0)
    ec = jnp.concatenate(emb_c, axis=0)
    h = (jnp.dot(ea, w1a_ref[...], preferred_element_type=jnp.float32)
         + jnp.dot(ec, w1c_ref[...], preferred_element_type=jnp.float32)
         + b1_ref[...])
    h = h * jax.nn.sigmoid(h)
    out = (jnp.dot(h, w2_ref[...], preferred_element_type=jnp.float32)
           + b2_ref[...])
    out_ref[...] = out.reshape(INV_BLK, 64, D_INV)


def _edges_body(bt_ref, tbl_ref, out_ref):
    tbl = tbl_ref[...]
    for i in range(EDGE_BLK):
        oh = _onehot_t(bt_ref[i:i + 1, :], B_VOCAB)
        out_ref[pl.ds(i * 64, 64), :] = lax.dot_general(
            oh, tbl, _TDOT, preferred_element_type=jnp.float32)


def kernel(atom_types, bond_types, atom_mask, atom_charges, atom_type_table,
           charge_table, bond_table, W1, b1, W2, b2):
    del atom_mask

    ta = jnp.pad(atom_type_table, ((0, A_VOCAB - atom_type_table.shape[0]), (0, 0)))
    tc = jnp.pad(charge_table, ((0, C_VOCAB - charge_table.shape[0]), (0, 0)))
    w1a, w1c = W1[:EMB], W1[EMB:]
    b1r = b1.reshape(1, D_INV)
    b2r = b2.reshape(1, D_INV)

    full = lambda shape: pl.BlockSpec(shape, lambda i: (0,) * len(shape))
    invs = pl.pallas_call(
        _invs_body,
        grid=(B // INV_BLK,),
        in_specs=[
            pl.BlockSpec((INV_BLK, 64), lambda i: (i, 0)),
            pl.BlockSpec((INV_BLK, 64), lambda i: (i, 0)),
            full((A_VOCAB, EMB)),
            full((C_VOCAB, EMB)),
            full((EMB, D_INV)),
            full((EMB, D_INV)),
            full((1, D_INV)),
            full((D_INV, D_INV)),
            full((1, D_INV)),
        ],
        out_specs=pl.BlockSpec((INV_BLK, 64, D_INV), lambda i: (i, 0, 0)),
        out_shape=jax.ShapeDtypeStruct((B, N, D_INV), jnp.float32),
    )(atom_types, atom_charges, ta, tc, w1a, w1c, b1r, W2, b2r)

    bt = bond_types.reshape(ROWS, N)
    edges = pl.pallas_call(
        _edges_body,
        grid=(ROWS // EDGE_BLK,),
        in_specs=[
            pl.BlockSpec((EDGE_BLK, 64), lambda i: (i, 0)),
            full((B_VOCAB, EMB)),
        ],
        out_specs=pl.BlockSpec((EDGE_BLK * 64, EMB), lambda i: (i, 0)),
        out_shape=jax.ShapeDtypeStruct((E_ROWS, EMB), jnp.float32),
    )(bt, bond_table)

    return invs, edges.reshape(B, N, N, EMB)
